# Initial kernel scaffold; baseline (speedup 1.0000x reference)
import jax
import jax.numpy as jnp
from jax.experimental import pallas as pl


def kernel(n, adj_mat, W0, att_src0, att_dst0, bias0, W1, att_src1, att_dst1, bias1):
    raise NotImplementedError("write your pallas kernel here")


# speedup vs baseline: 2.1676x; 2.1676x over previous
import functools

import jax
import jax.numpy as jnp
from jax import lax
from jax.experimental import pallas as pl
from jax.experimental.pallas import tpu as pltpu


def _gat2_body(x_ref, adj_ref, w0_ref, as0_ref, ad0_ref, b0_ref,
               w1_ref, as1_ref, ad1_ref, b1_ref, o_ref, *, n_nodes, heads, dk):
    x = x_ref[0]
    adj = adj_ref[0]
    row = lax.broadcasted_iota(jnp.int32, (n_nodes, n_nodes), 0)
    col = lax.broadcasted_iota(jnp.int32, (n_nodes, n_nodes), 1)
    mask = jnp.logical_or(adj != 0.0, row == col)

    def gat(xin, w_ref, asel_ref, adel_ref, b_ref):
        h = jnp.dot(xin, w_ref[...], preferred_element_type=jnp.float32)
        ad_mat = jnp.dot(h, adel_ref[...], preferred_element_type=jnp.float32)
        ast = lax.dot_general(asel_ref[...], h,
                              (((0,), (1,)), ((), ())),
                              preferred_element_type=jnp.float32)
        outs = []
        for hi in range(heads):
            e = ad_mat[:, hi:hi + 1] + ast[hi:hi + 1, :]
            e = jnp.where(e >= 0.0, e, 0.2 * e)
            e = jnp.where(mask, e, jnp.float32(-1e9))
            m = jnp.max(e, axis=1, keepdims=True)
            p = jnp.exp(e - m)
            s = jnp.sum(p, axis=1, keepdims=True)
            attn = p / s
            outs.append(jnp.dot(attn, h[:, hi * dk:(hi + 1) * dk],
                                preferred_element_type=jnp.float32))
        return jnp.concatenate(outs, axis=1) + b_ref[...]

    x1 = jnp.maximum(gat(x, w0_ref, as0_ref, ad0_ref, b0_ref), 0.0)
    x2 = jnp.maximum(gat(x1, w1_ref, as1_ref, ad1_ref, b1_ref), 0.0)
    o_ref[0] = x2


def _head_selector(att, heads, dk):
    hdk = heads * dk
    flat = att.reshape(hdk)
    rows = jnp.arange(hdk)
    onehot = (rows[:, None] // dk == jnp.arange(heads)[None, :]).astype(att.dtype)
    return onehot * flat[:, None]


def kernel(n, adj_mat, W0, att_src0, att_dst0, bias0,
           W1, att_src1, att_dst1, bias1):
    b, nn, d = n.shape
    heads, dk = att_src0.shape
    hdk = heads * dk

    as0 = _head_selector(att_src0, heads, dk)
    ad0 = _head_selector(att_dst0, heads, dk)
    as1 = _head_selector(att_src1, heads, dk)
    ad1 = _head_selector(att_dst1, heads, dk)
    b0 = bias0.reshape(1, hdk)
    b1 = bias1.reshape(1, hdk)

    body = functools.partial(_gat2_body, n_nodes=nn, heads=heads, dk=dk)
    full = lambda shape: pl.BlockSpec(shape, lambda i: (0,) * len(shape))
    out = pl.pallas_call(
        body,
        grid=(b,),
        in_specs=[
            pl.BlockSpec((1, nn, d), lambda i: (i, 0, 0)),
            pl.BlockSpec((1, nn, nn), lambda i: (i, 0, 0)),
            full((d, hdk)), full((hdk, heads)), full((hdk, heads)), full((1, hdk)),
            full((d, hdk)), full((hdk, heads)), full((hdk, heads)), full((1, hdk)),
        ],
        out_specs=pl.BlockSpec((1, nn, hdk), lambda i: (i, 0, 0)),
        out_shape=jax.ShapeDtypeStruct((b, nn, hdk), jnp.float32),
        compiler_params=pltpu.CompilerParams(
            dimension_semantics=("parallel",)),
    )(n, adj_mat, W0, as0, ad0, b0, W1, as1, ad1, b1)
    return out
